# baseline (speedup 1.0000x reference)
import jax
import jax.numpy as jnp
from jax import lax
from jax.experimental import pallas as pl
from jax.experimental.pallas import tpu as pltpu
from jax.experimental.pallas import tpu_sc as plsc

_B = 4096
_D = 128
_K = 2
_N = 32

_BT = 3584
_BS = _B - _BT

_R = _B * _K
_ROFF = _BT * _K
_RSC = _BS * _K
_NW = 32
_RPW = _RSC // _NW
_BPW = _RPW // _K
_CH = 8
_CB = _CH // _K
_NCHUNK = _RPW // _CH

_BB = 256
_BBM = 512


def _sc_pool_body(nbr_hbm, m_hbm, out0_hbm, out1_hbm,
                  buf0, buf1, mball, ob0a, ob0b, ob1a, ob1b,
                  sem0, sem1, msem, osem0, osem1):
    c = lax.axis_index("c")
    s = lax.axis_index("s")
    wid = s * 2 + c
    row0 = _ROFF + wid * _RPW
    bat0 = wid * _BPW
    bufs = [buf0, buf1]
    sems = [sem0, sem1]
    obs = [(ob0a, ob1a), (ob0b, ob1b)]
    osems = [osem0, osem1]

    pltpu.async_copy(m_hbm.at[pl.ds(row0, _RPW)], mball, msem)

    def issue(g, b):
        pltpu.async_copy(nbr_hbm.at[pl.ds(row0 + g * _CH, _CH)], bufs[b], sems[b])

    def wait_in(b):
        pltpu.make_async_copy(nbr_hbm.at[pl.ds(0, _CH)], bufs[b], sems[b]).wait()

    issue(0, 0)
    pltpu.make_async_copy(m_hbm.at[pl.ds(0, _RPW)], mball, msem).wait()

    def pair(p, carry):
        for b in range(2):
            g = 2 * p + b
            wait_in(b)

            @pl.when(g + 1 < _NCHUNK)
            def _():
                issue(g + 1, 1 - b)

            @pl.when(p >= 1)
            def _():
                pltpu.make_async_copy(obs[b][0], out0_hbm.at[pl.ds(0, _CB)],
                                      osems[b]).wait()
                pltpu.make_async_copy(obs[b][1], out1_hbm.at[pl.ds(0, _CB)],
                                      osems[b]).wait()

            buf = bufs[b]
            o0, o1 = obs[b]

            def row2(i, carry2):
                for kk in range(_K):
                    ii = _K * i + kk
                    acc = [jnp.zeros((16,), jnp.float32) for _ in range(8)]
                    mrow = g * _CH + ii
                    mv0 = mball[mrow, pl.ds(0, 16)]
                    mv1 = mball[mrow, pl.ds(16, 16)]
                    for n in range(_N):
                        mn = mv0[n] if n < 16 else mv1[n - 16]
                        for j in range(8):
                            acc[j] = acc[j] + mn * buf[ii, n, pl.ds(j * 16, 16)]
                    dst = o0 if kk == 0 else o1
                    for j in range(8):
                        dst[i, pl.ds(j * 16, 16)] = acc[j]
                return carry2

            lax.fori_loop(0, _CB, row2, 0)

            base = bat0 + g * _CB
            pltpu.async_copy(o0, out0_hbm.at[pl.ds(base, _CB)], osems[b])
            pltpu.async_copy(o1, out1_hbm.at[pl.ds(base, _CB)], osems[b])
        return carry

    lax.fori_loop(0, _NCHUNK // 2, pair, 0)
    for b in range(2):
        pltpu.make_async_copy(obs[b][0], out0_hbm.at[pl.ds(0, _CB)],
                              osems[b]).wait()
        pltpu.make_async_copy(obs[b][1], out1_hbm.at[pl.ds(0, _CB)],
                              osems[b]).wait()


def _sc_pool(nbr3, m2):
    mesh = plsc.VectorSubcoreMesh(core_axis_name="c", subcore_axis_name="s")
    f = pl.kernel(
        _sc_pool_body,
        mesh=mesh,
        out_type=(
            jax.ShapeDtypeStruct((_BS, _D), jnp.float32),
            jax.ShapeDtypeStruct((_BS, _D), jnp.float32),
        ),
        scratch_types=[
            pltpu.VMEM((_CH, _N, _D), jnp.float32),
            pltpu.VMEM((_CH, _N, _D), jnp.float32),
            pltpu.VMEM((_RPW, _N), jnp.float32),
            pltpu.VMEM((_CB, _D), jnp.float32),
            pltpu.VMEM((_CB, _D), jnp.float32),
            pltpu.VMEM((_CB, _D), jnp.float32),
            pltpu.VMEM((_CB, _D), jnp.float32),
            pltpu.SemaphoreType.DMA,
            pltpu.SemaphoreType.DMA,
            pltpu.SemaphoreType.DMA,
            pltpu.SemaphoreType.DMA,
            pltpu.SemaphoreType.DMA,
        ],
    )
    return f(nbr3, m2)


def _fused_body(nbr_ref, m_ref, sv_ref, wt_ref, b_ref, out_ref):
    nbr = nbr_ref[...]
    m = m_ref[...]
    e = jnp.sum(nbr * m[..., None], axis=2)
    scale = jnp.float32(1.0 / _N)
    x0 = sv_ref[...]
    e0 = e[:, 0, :] * scale
    e1 = e[:, 1, :] * scale
    acc = jnp.dot(x0, wt_ref[0:_D, :], preferred_element_type=jnp.float32)
    acc += jnp.dot(e0, wt_ref[_D:2 * _D, :], preferred_element_type=jnp.float32)
    acc += jnp.dot(e1, wt_ref[2 * _D:3 * _D, :], preferred_element_type=jnp.float32)
    out_ref[...] = acc + b_ref[...]


def _tc_fused(nbr, m, sv, wt, bb):
    grid = (_BT // _BB,)
    return pl.pallas_call(
        _fused_body,
        grid=grid,
        in_specs=[
            pl.BlockSpec((_BB, _K, _N, _D), lambda i: (i, 0, 0, 0)),
            pl.BlockSpec((_BB, _K, _N), lambda i: (i, 0, 0)),
            pl.BlockSpec((_BB, _D), lambda i: (i, 0)),
            pl.BlockSpec((3 * _D, _D), lambda i: (0, 0)),
            pl.BlockSpec((1, _D), lambda i: (0, 0)),
        ],
        out_specs=pl.BlockSpec((_BB, _D), lambda i: (i, 0)),
        out_shape=jax.ShapeDtypeStruct((_BT, _D), jnp.float32),
        compiler_params=pltpu.CompilerParams(
            dimension_semantics=("arbitrary",),
        ),
    )(nbr, m, sv, wt, bb)


def _mm_body(e0_ref, e1_ref, sv_ref, wt_ref, b_ref, out_ref):
    scale = jnp.float32(1.0 / _N)
    x0 = sv_ref[...]
    e0 = e0_ref[...] * scale
    e1 = e1_ref[...] * scale
    acc = jnp.dot(x0, wt_ref[0:_D, :], preferred_element_type=jnp.float32)
    acc += jnp.dot(e0, wt_ref[_D:2 * _D, :], preferred_element_type=jnp.float32)
    acc += jnp.dot(e1, wt_ref[2 * _D:3 * _D, :], preferred_element_type=jnp.float32)
    out_ref[...] = acc + b_ref[...]


def _tc_matmul(e0, e1, sv, wt, bb):
    grid = (_BS // _BBM,)
    off = _BT // _BBM
    return pl.pallas_call(
        _mm_body,
        grid=grid,
        in_specs=[
            pl.BlockSpec((_BBM, _D), lambda i: (i, 0)),
            pl.BlockSpec((_BBM, _D), lambda i: (i, 0)),
            pl.BlockSpec((_BBM, _D), lambda i: (i + off, 0)),
            pl.BlockSpec((3 * _D, _D), lambda i: (0, 0)),
            pl.BlockSpec((1, _D), lambda i: (0, 0)),
        ],
        out_specs=pl.BlockSpec((_BBM, _D), lambda i: (i, 0)),
        out_shape=jax.ShapeDtypeStruct((_BS, _D), jnp.float32),
        compiler_params=pltpu.CompilerParams(
            dimension_semantics=("arbitrary",),
        ),
    )(e0, e1, sv, wt, bb)


def kernel(self_vectors, neighbor_vectors, masks, W, b):
    nbr4 = neighbor_vectors.reshape(_B, _K, _N, _D)
    nbr3 = neighbor_vectors.reshape(_R, _N, _D)
    m3 = masks.reshape(_B, _K, _N)
    m2 = masks.reshape(_R, _N)
    sv = self_vectors.reshape(_B, _D)
    wt = W.T
    bb = b.reshape(1, _D)

    e0, e1 = _sc_pool(nbr3, m2)
    out_tc = _tc_fused(nbr4, m3, sv, wt, bb)
    out_sc = _tc_matmul(e0, e1, sv, wt, bb)
    out = jnp.concatenate([out_tc, out_sc], axis=0)
    return out.reshape(_B, 1, _D)


# speedup vs baseline: 1.0088x; 1.0088x over previous
import jax
import jax.numpy as jnp
from jax import lax
from jax.experimental import pallas as pl
from jax.experimental.pallas import tpu as pltpu
from jax.experimental.pallas import tpu_sc as plsc

_B = 4096
_D = 128
_K = 2
_N = 32

_BT = 3072
_BS = _B - _BT

_R = _B * _K
_ROFF = _BT * _K
_RSC = _BS * _K
_NW = 32
_RPW = _RSC // _NW
_BPW = _RPW // _K
_CH = 8
_CB = _CH // _K
_NCHUNK = _RPW // _CH

_BB = 256
_BBM = 512


def _sc_pool_body(nbr_hbm, m_hbm, out0_hbm, out1_hbm,
                  buf0, buf1, mball, ob0a, ob0b, ob1a, ob1b,
                  sem0, sem1, msem, osem0, osem1):
    c = lax.axis_index("c")
    s = lax.axis_index("s")
    wid = s * 2 + c
    row0 = _ROFF + wid * _RPW
    bat0 = wid * _BPW
    bufs = [buf0, buf1]
    sems = [sem0, sem1]
    obs = [(ob0a, ob1a), (ob0b, ob1b)]
    osems = [osem0, osem1]

    pltpu.async_copy(m_hbm.at[pl.ds(row0, _RPW)], mball, msem)

    def issue(g, b):
        pltpu.async_copy(nbr_hbm.at[pl.ds(row0 + g * _CH, _CH)], bufs[b], sems[b])

    def wait_in(b):
        pltpu.make_async_copy(nbr_hbm.at[pl.ds(0, _CH)], bufs[b], sems[b]).wait()

    issue(0, 0)
    pltpu.make_async_copy(m_hbm.at[pl.ds(0, _RPW)], mball, msem).wait()

    def pair(p, carry):
        for b in range(2):
            g = 2 * p + b
            wait_in(b)

            @pl.when(g + 1 < _NCHUNK)
            def _():
                issue(g + 1, 1 - b)

            @pl.when(p >= 1)
            def _():
                pltpu.make_async_copy(obs[b][0], out0_hbm.at[pl.ds(0, _CB)],
                                      osems[b]).wait()
                pltpu.make_async_copy(obs[b][1], out1_hbm.at[pl.ds(0, _CB)],
                                      osems[b]).wait()

            buf = bufs[b]
            o0, o1 = obs[b]

            def row2(i, carry2):
                for kk in range(_K):
                    ii = _K * i + kk
                    acc = [jnp.zeros((16,), jnp.float32) for _ in range(8)]
                    mrow = g * _CH + ii
                    mv0 = mball[mrow, pl.ds(0, 16)]
                    mv1 = mball[mrow, pl.ds(16, 16)]
                    for n in range(_N):
                        mn = mv0[n] if n < 16 else mv1[n - 16]
                        for j in range(8):
                            acc[j] = acc[j] + mn * buf[ii, n, pl.ds(j * 16, 16)]
                    dst = o0 if kk == 0 else o1
                    for j in range(8):
                        dst[i, pl.ds(j * 16, 16)] = acc[j]
                return carry2

            lax.fori_loop(0, _CB, row2, 0)

            base = bat0 + g * _CB
            pltpu.async_copy(o0, out0_hbm.at[pl.ds(base, _CB)], osems[b])
            pltpu.async_copy(o1, out1_hbm.at[pl.ds(base, _CB)], osems[b])
        return carry

    lax.fori_loop(0, _NCHUNK // 2, pair, 0)
    for b in range(2):
        pltpu.make_async_copy(obs[b][0], out0_hbm.at[pl.ds(0, _CB)],
                              osems[b]).wait()
        pltpu.make_async_copy(obs[b][1], out1_hbm.at[pl.ds(0, _CB)],
                              osems[b]).wait()


def _sc_pool(nbr3, m2):
    mesh = plsc.VectorSubcoreMesh(core_axis_name="c", subcore_axis_name="s")
    f = pl.kernel(
        _sc_pool_body,
        mesh=mesh,
        out_type=(
            jax.ShapeDtypeStruct((_BS, _D), jnp.float32),
            jax.ShapeDtypeStruct((_BS, _D), jnp.float32),
        ),
        scratch_types=[
            pltpu.VMEM((_CH, _N, _D), jnp.float32),
            pltpu.VMEM((_CH, _N, _D), jnp.float32),
            pltpu.VMEM((_RPW, _N), jnp.float32),
            pltpu.VMEM((_CB, _D), jnp.float32),
            pltpu.VMEM((_CB, _D), jnp.float32),
            pltpu.VMEM((_CB, _D), jnp.float32),
            pltpu.VMEM((_CB, _D), jnp.float32),
            pltpu.SemaphoreType.DMA,
            pltpu.SemaphoreType.DMA,
            pltpu.SemaphoreType.DMA,
            pltpu.SemaphoreType.DMA,
            pltpu.SemaphoreType.DMA,
        ],
    )
    return f(nbr3, m2)


def _fused_body(nbr_ref, m_ref, sv_ref, wt_ref, b_ref, out_ref):
    nbr = nbr_ref[...]
    m = m_ref[...]
    e = jnp.sum(nbr * m[..., None], axis=2)
    scale = jnp.float32(1.0 / _N)
    x0 = sv_ref[...]
    e0 = e[:, 0, :] * scale
    e1 = e[:, 1, :] * scale
    acc = jnp.dot(x0, wt_ref[0:_D, :], preferred_element_type=jnp.float32)
    acc += jnp.dot(e0, wt_ref[_D:2 * _D, :], preferred_element_type=jnp.float32)
    acc += jnp.dot(e1, wt_ref[2 * _D:3 * _D, :], preferred_element_type=jnp.float32)
    out_ref[...] = acc + b_ref[...]


def _tc_fused(nbr, m, sv, wt, bb):
    grid = (_BT // _BB,)
    return pl.pallas_call(
        _fused_body,
        grid=grid,
        in_specs=[
            pl.BlockSpec((_BB, _K, _N, _D), lambda i: (i, 0, 0, 0)),
            pl.BlockSpec((_BB, _K, _N), lambda i: (i, 0, 0)),
            pl.BlockSpec((_BB, _D), lambda i: (i, 0)),
            pl.BlockSpec((3 * _D, _D), lambda i: (0, 0)),
            pl.BlockSpec((1, _D), lambda i: (0, 0)),
        ],
        out_specs=pl.BlockSpec((_BB, _D), lambda i: (i, 0)),
        out_shape=jax.ShapeDtypeStruct((_BT, _D), jnp.float32),
        compiler_params=pltpu.CompilerParams(
            dimension_semantics=("arbitrary",),
        ),
    )(nbr, m, sv, wt, bb)


def _mm_body(e0_ref, e1_ref, sv_ref, wt_ref, b_ref, out_ref):
    scale = jnp.float32(1.0 / _N)
    x0 = sv_ref[...]
    e0 = e0_ref[...] * scale
    e1 = e1_ref[...] * scale
    acc = jnp.dot(x0, wt_ref[0:_D, :], preferred_element_type=jnp.float32)
    acc += jnp.dot(e0, wt_ref[_D:2 * _D, :], preferred_element_type=jnp.float32)
    acc += jnp.dot(e1, wt_ref[2 * _D:3 * _D, :], preferred_element_type=jnp.float32)
    out_ref[...] = acc + b_ref[...]


def _tc_matmul(e0, e1, sv, wt, bb):
    grid = (_BS // _BBM,)
    off = _BT // _BBM
    return pl.pallas_call(
        _mm_body,
        grid=grid,
        in_specs=[
            pl.BlockSpec((_BBM, _D), lambda i: (i, 0)),
            pl.BlockSpec((_BBM, _D), lambda i: (i, 0)),
            pl.BlockSpec((_BBM, _D), lambda i: (i + off, 0)),
            pl.BlockSpec((3 * _D, _D), lambda i: (0, 0)),
            pl.BlockSpec((1, _D), lambda i: (0, 0)),
        ],
        out_specs=pl.BlockSpec((_BBM, _D), lambda i: (i, 0)),
        out_shape=jax.ShapeDtypeStruct((_BS, _D), jnp.float32),
        compiler_params=pltpu.CompilerParams(
            dimension_semantics=("arbitrary",),
        ),
    )(e0, e1, sv, wt, bb)


def kernel(self_vectors, neighbor_vectors, masks, W, b):
    nbr4 = neighbor_vectors.reshape(_B, _K, _N, _D)
    nbr3 = neighbor_vectors.reshape(_R, _N, _D)
    m3 = masks.reshape(_B, _K, _N)
    m2 = masks.reshape(_R, _N)
    sv = self_vectors.reshape(_B, _D)
    wt = W.T
    bb = b.reshape(1, _D)

    e0, e1 = _sc_pool(nbr3, m2)
    out_tc = _tc_fused(nbr4, m3, sv, wt, bb)
    out_sc = _tc_matmul(e0, e1, sv, wt, bb)
    out = jnp.concatenate([out_tc, out_sc], axis=0)
    return out.reshape(_B, 1, _D)


# speedup vs baseline: 1.0448x; 1.0357x over previous
import jax
import jax.numpy as jnp
from jax import lax
from jax.experimental import pallas as pl
from jax.experimental.pallas import tpu as pltpu
from jax.experimental.pallas import tpu_sc as plsc

_B = 4096
_D = 128
_K = 2
_N = 32

_BT = 3072
_BS = _B - _BT

_R = _B * _K
_ROFF = _BT * _K
_RSC = _BS * _K
_NW = 32
_RPW = _RSC // _NW
_BPW = _RPW // _K
_CH = 8
_CB = _CH // _K
_NCHUNK = _RPW // _CH

_BB = 256
_BBM = 512


def _sc_pool_body(nbr_hbm, m_hbm, out0_hbm, out1_hbm,
                  buf0, buf1, mball, ob0a, ob0b, ob1a, ob1b,
                  sem0, sem1, msem, osem0, osem1):
    c = lax.axis_index("c")
    s = lax.axis_index("s")
    wid = s * 2 + c
    row0 = _ROFF + wid * _RPW
    bat0 = wid * _BPW
    bufs = [buf0, buf1]
    sems = [sem0, sem1]
    obs = [(ob0a, ob1a), (ob0b, ob1b)]
    osems = [osem0, osem1]

    pltpu.async_copy(m_hbm.at[pl.ds(row0, _RPW)], mball, msem)

    def issue(g, b):
        pltpu.async_copy(nbr_hbm.at[pl.ds(row0 + g * _CH, _CH)], bufs[b], sems[b])

    def wait_in(b):
        pltpu.make_async_copy(nbr_hbm.at[pl.ds(0, _CH)], bufs[b], sems[b]).wait()

    issue(0, 0)
    pltpu.make_async_copy(m_hbm.at[pl.ds(0, _RPW)], mball, msem).wait()

    def pair(p, carry):
        for b in range(2):
            g = 2 * p + b
            wait_in(b)

            @pl.when(g + 1 < _NCHUNK)
            def _():
                issue(g + 1, 1 - b)

            @pl.when(p >= 1)
            def _():
                pltpu.make_async_copy(obs[b][0], out0_hbm.at[pl.ds(0, _CB)],
                                      osems[b]).wait()
                pltpu.make_async_copy(obs[b][1], out1_hbm.at[pl.ds(0, _CB)],
                                      osems[b]).wait()

            buf = bufs[b]
            o0, o1 = obs[b]

            def row2(i, carry2):
                for kk in range(_K):
                    ii = _K * i + kk
                    acc = [jnp.zeros((16,), jnp.float32) for _ in range(8)]
                    mrow = g * _CH + ii
                    mv0 = mball[mrow, pl.ds(0, 16)]
                    mv1 = mball[mrow, pl.ds(16, 16)]
                    for n in range(_N):
                        mn = mv0[n] if n < 16 else mv1[n - 16]
                        for j in range(8):
                            acc[j] = acc[j] + mn * buf[ii, n, pl.ds(j * 16, 16)]
                    dst = o0 if kk == 0 else o1
                    for j in range(8):
                        dst[i, pl.ds(j * 16, 16)] = acc[j]
                return carry2

            lax.fori_loop(0, _CB, row2, 0)

            base = bat0 + g * _CB
            pltpu.async_copy(o0, out0_hbm.at[pl.ds(base, _CB)], osems[b])
            pltpu.async_copy(o1, out1_hbm.at[pl.ds(base, _CB)], osems[b])
        return carry

    lax.fori_loop(0, _NCHUNK // 2, pair, 0)
    for b in range(2):
        pltpu.make_async_copy(obs[b][0], out0_hbm.at[pl.ds(0, _CB)],
                              osems[b]).wait()
        pltpu.make_async_copy(obs[b][1], out1_hbm.at[pl.ds(0, _CB)],
                              osems[b]).wait()


def _sc_pool(nbr3, m2):
    mesh = plsc.VectorSubcoreMesh(core_axis_name="c", subcore_axis_name="s")
    f = pl.kernel(
        _sc_pool_body,
        mesh=mesh,
        out_type=(
            jax.ShapeDtypeStruct((_BS, _D), jnp.float32),
            jax.ShapeDtypeStruct((_BS, _D), jnp.float32),
        ),
        scratch_types=[
            pltpu.VMEM((_CH, _N, _D), jnp.float32),
            pltpu.VMEM((_CH, _N, _D), jnp.float32),
            pltpu.VMEM((_RPW, _N), jnp.float32),
            pltpu.VMEM((_CB, _D), jnp.float32),
            pltpu.VMEM((_CB, _D), jnp.float32),
            pltpu.VMEM((_CB, _D), jnp.float32),
            pltpu.VMEM((_CB, _D), jnp.float32),
            pltpu.SemaphoreType.DMA,
            pltpu.SemaphoreType.DMA,
            pltpu.SemaphoreType.DMA,
            pltpu.SemaphoreType.DMA,
            pltpu.SemaphoreType.DMA,
        ],
    )
    return f(nbr3, m2)


def _fused_body(nbr_ref, m_ref, sv_ref, wt_ref, b_ref, out_ref):
    nbr = nbr_ref[...]
    m = m_ref[...]
    e = jnp.sum(nbr * m[..., None], axis=2)
    scale = jnp.float32(1.0 / _N)
    x0 = sv_ref[...]
    e0 = e[:, 0, :] * scale
    e1 = e[:, 1, :] * scale
    acc = jnp.dot(x0, wt_ref[0:_D, :], preferred_element_type=jnp.float32)
    acc += jnp.dot(e0, wt_ref[_D:2 * _D, :], preferred_element_type=jnp.float32)
    acc += jnp.dot(e1, wt_ref[2 * _D:3 * _D, :], preferred_element_type=jnp.float32)
    out_ref[...] = acc + b_ref[...]


def _tc_fused(nbr, m, sv, wt, bb):
    grid = (_BT // _BB,)
    return pl.pallas_call(
        _fused_body,
        grid=grid,
        in_specs=[
            pl.BlockSpec((_BB, _K, _N, _D), lambda i: (i, 0, 0, 0)),
            pl.BlockSpec((_BB, _K, _N), lambda i: (i, 0, 0)),
            pl.BlockSpec((_BB, _D), lambda i: (i, 0)),
            pl.BlockSpec((3 * _D, _D), lambda i: (0, 0)),
            pl.BlockSpec((1, _D), lambda i: (0, 0)),
        ],
        out_specs=pl.BlockSpec((_BB, _D), lambda i: (i, 0)),
        out_shape=jax.ShapeDtypeStruct((_B, _D), jnp.float32),
        compiler_params=pltpu.CompilerParams(
            dimension_semantics=("arbitrary",),
        ),
    )(nbr, m, sv, wt, bb)


def _mm_body(e0_ref, e1_ref, sv_ref, wt_ref, b_ref, head_ref, out_ref):
    del head_ref
    scale = jnp.float32(1.0 / _N)
    x0 = sv_ref[...]
    e0 = e0_ref[...] * scale
    e1 = e1_ref[...] * scale
    acc = jnp.dot(x0, wt_ref[0:_D, :], preferred_element_type=jnp.float32)
    acc += jnp.dot(e0, wt_ref[_D:2 * _D, :], preferred_element_type=jnp.float32)
    acc += jnp.dot(e1, wt_ref[2 * _D:3 * _D, :], preferred_element_type=jnp.float32)
    out_ref[...] = acc + b_ref[...]


def _tc_matmul(e0, e1, sv, wt, bb, head):
    grid = (_BS // _BBM,)
    off = _BT // _BBM
    return pl.pallas_call(
        _mm_body,
        grid=grid,
        in_specs=[
            pl.BlockSpec((_BBM, _D), lambda i: (i, 0)),
            pl.BlockSpec((_BBM, _D), lambda i: (i, 0)),
            pl.BlockSpec((_BBM, _D), lambda i: (i + off, 0)),
            pl.BlockSpec((3 * _D, _D), lambda i: (0, 0)),
            pl.BlockSpec((1, _D), lambda i: (0, 0)),
            pl.BlockSpec(memory_space=pl.MemorySpace.ANY),
        ],
        out_specs=pl.BlockSpec((_BBM, _D), lambda i: (i + off, 0)),
        out_shape=jax.ShapeDtypeStruct((_B, _D), jnp.float32),
        input_output_aliases={5: 0},
        compiler_params=pltpu.CompilerParams(
            dimension_semantics=("arbitrary",),
        ),
    )(e0, e1, sv, wt, bb, head)


def kernel(self_vectors, neighbor_vectors, masks, W, b):
    nbr4 = neighbor_vectors.reshape(_B, _K, _N, _D)
    nbr3 = neighbor_vectors.reshape(_R, _N, _D)
    m3 = masks.reshape(_B, _K, _N)
    m2 = masks.reshape(_R, _N)
    sv = self_vectors.reshape(_B, _D)
    wt = W.T
    bb = b.reshape(1, _D)

    e0, e1 = _sc_pool(nbr3, m2)
    head = _tc_fused(nbr4, m3, sv, wt, bb)
    out = _tc_matmul(e0, e1, sv, wt, bb, head)
    return out.reshape(_B, 1, _D)
